# baseline (speedup 1.0000x reference)
import jax
import jax.numpy as jnp
from jax import lax
from jax.experimental import pallas as pl
from jax.experimental.pallas import tpu as pltpu
from jax.experimental.pallas import tpu_sc as plsc

N_FACTORS = 32
BATCH = 16384
NC = 2
NS = 16
NW = NC * NS
L = 16
B_PER_W = BATCH // NW
CHUNK = 128
NCHUNK = B_PER_W // CHUNK


def _body(users_hbm, items_hbm, uf_hbm, vf_hbm, out_hbm,
          uidx_v, iidx_v, urows_v, vrows_v, out_v, sem):
  wid = lax.axis_index("s") * NC + lax.axis_index("c")

  pltpu.sync_copy(users_hbm.at[wid], uidx_v)
  pltpu.sync_copy(items_hbm.at[wid], iidx_v)

  copies = []
  for j in range(NCHUNK):
    copies.append(pltpu.async_copy(
        uf_hbm.at[uidx_v.at[j]], urows_v.at[pl.ds(j * CHUNK, CHUNK)], sem))
    copies.append(pltpu.async_copy(
        vf_hbm.at[iidx_v.at[j]], vrows_v.at[pl.ds(j * CHUNK, CHUNK)], sem))
  for c in copies:
    c.wait()

  lanes = lax.iota(jnp.int32, L)

  def group(g, carry):
    acc = jnp.zeros((L,), jnp.float32)
    for j in range(L):
      r = g * L + j
      u0 = urows_v[r, pl.ds(0, L)]
      u1 = urows_v[r, pl.ds(L, L)]
      v0 = vrows_v[r, pl.ds(0, L)]
      v1 = vrows_v[r, pl.ds(L, L)]
      s = u0 * v0 + u1 * v1
      acc = jnp.where(lanes == j, jnp.sum(s), acc)
    out_v[pl.ds(g * L, L)] = acc
    return carry

  lax.fori_loop(0, B_PER_W // L, group, 0)

  pltpu.sync_copy(out_v, out_hbm.at[pl.ds(wid * B_PER_W, B_PER_W)])


@jax.jit
def kernel(data, user_factors, item_factors):
  users = data[:, 0].reshape(NW, NCHUNK, CHUNK)
  items = data[:, 1].reshape(NW, NCHUNK, CHUNK)
  mesh = plsc.VectorSubcoreMesh(
      core_axis_name="c", subcore_axis_name="s", num_cores=NC,
      num_subcores=NS)
  run = pl.kernel(
      _body,
      out_type=jax.ShapeDtypeStruct((BATCH,), jnp.float32),
      mesh=mesh,
      compiler_params=pltpu.CompilerParams(
          needs_layout_passes=False, use_tc_tiling_on_sc=False),
      scratch_types=[
          pltpu.VMEM((NCHUNK, CHUNK), jnp.int32),
          pltpu.VMEM((NCHUNK, CHUNK), jnp.int32),
          pltpu.VMEM((B_PER_W, N_FACTORS), jnp.float32),
          pltpu.VMEM((B_PER_W, N_FACTORS), jnp.float32),
          pltpu.VMEM((B_PER_W,), jnp.float32),
          pltpu.SemaphoreType.DMA,
      ],
  )
  return run(users, items, user_factors, item_factors)
